# baseline (speedup 1.0000x reference)
import functools

import jax
import jax.numpy as jnp
from jax.experimental import pallas as pl
from jax.experimental.pallas import tpu as pltpu

_D = 128
_M = 64
_BI = 128
_BJ = 128


def _silu(x):
    return x * (0.5 * jnp.tanh(0.5 * x) + 0.5)


def _dot(a, b, dims=(((1,), (0,)), ((), ()))):
    return jax.lax.dot_general(a, b, dims, preferred_element_type=jnp.float32)


def _egnn_body(n, npad, njt, fi_ref, fj_ref, ci_ref, cj_ref, e_ref, wi_ref,
               wj_ref, w2t_ref, sm_ref, w1f_ref, w1a_ref, wh2_ref, nb_ref,
               out_ref, acc_ref):
    j = pl.program_id(1)

    fi = fi_ref[...]
    fj = fj_ref[...]
    ci = ci_ref[...]
    cj = cj_ref[...]
    ee = e_ref[...]
    wdc = sm_ref[:, 0:1]
    wec = sm_ref[:, 1:2]
    be1c = sm_ref[:, 2:3]
    be2c = sm_ref[:, 3:4]

    ti = _dot(fi, wi_ref[...])
    tjT = jax.lax.dot_general(wj_ref[...], fj, (((0,), (1,)), ((), ())),
                              preferred_element_type=jnp.float32)
    qi = jnp.sum(ci * ci, axis=1, keepdims=True)
    qjT = jax.lax.dot_general(jnp.ones((1, _D), jnp.float32), cj * cj,
                              (((1,), (1,)), ((), ())),
                              preferred_element_type=jnp.float32)
    cross = jax.lax.dot_general(ci, cj, (((1,), (1,)), ((), ())),
                                preferred_element_type=jnp.float32)
    vT = tjT + wdc * qjT + be1c

    w3b = jnp.concatenate([
        jnp.broadcast_to((-2.0 * wdc)[None], (_BI, _M, 1)),
        jnp.broadcast_to(wec[None], (_BI, _M, 1)),
        ti[:, :, None],
        jnp.broadcast_to(wdc[None], (_BI, _M, 1)),
    ], axis=2)
    s3 = jnp.concatenate([
        cross[:, None, :],
        ee[:, None, :],
        jnp.ones((_BI, 1, _BJ), jnp.float32),
        jnp.broadcast_to(qi[:, :, None], (_BI, 1, _BJ)),
    ], axis=1)
    m = jax.lax.dot_general(w3b, s3, (((2,), (1,)), ((0,), (0,))),
                            preferred_element_type=jnp.float32)
    m = _silu(m + vT[None, :, :])
    w2tb = jnp.broadcast_to(w2t_ref[...][None], (_BI, _M, _M))
    m = jax.lax.dot_general(w2tb, m, (((2,), (1,)), ((0,), (0,))),
                            preferred_element_type=jnp.float32)
    m = _silu(m + be2c[None, :, :])

    if npad != n:
        @pl.when(j == njt - 1)
        def _mask():
            jj = (j * _BJ
                  + jax.lax.broadcasted_iota(jnp.int32, (1, 1, _BJ), 2))
            acc_ref[...] += jnp.sum(jnp.where(jj < n, m, 0.0), axis=2)

        @pl.when(j == 0)
        def _first():
            acc_ref[...] = jnp.sum(m, axis=2)

        @pl.when(jnp.logical_and(j > 0, j < njt - 1))
        def _mid():
            acc_ref[...] += jnp.sum(m, axis=2)
    else:
        @pl.when(j == 0)
        def _first():
            acc_ref[...] = jnp.sum(m, axis=2)

        @pl.when(j > 0)
        def _mid():
            acc_ref[...] += jnp.sum(m, axis=2)

    @pl.when(j == njt - 1)
    def _last():
        f = fi_ref[...]
        agg = acc_ref[...]
        h1 = _silu(_dot(f, w1f_ref[...]) + _dot(agg, w1a_ref[...])
                   + nb_ref[0:1, :])
        out = _dot(h1, wh2_ref[...]) + nb_ref[1:2, :]
        out_ref[...] = jnp.maximum(f + out, 0.0)


def _egnn_apply(p, f, c, e, n):
    npad = -(-n // _BI) * _BI
    nit = npad // _BI
    njt = npad // _BJ
    fp = jnp.pad(f, ((0, npad - n), (0, 0)))
    cp = jnp.pad(c, ((0, npad - n), (0, _D - c.shape[1])))
    epd = jnp.pad(e, ((0, npad - n), (0, npad - n)))
    wi = p['We1'][:_D]
    wj = p['We1'][_D:2 * _D]
    sm = jnp.zeros((_M, 8), jnp.float32)
    sm = sm.at[:, 0].set(p['We1'][2 * _D])
    sm = sm.at[:, 1].set(p['We1'][2 * _D + 1])
    sm = sm.at[:, 2].set(p['be1'])
    sm = sm.at[:, 3].set(p['be2'])
    w2t = p['We2'].T
    w1f = p['Wh1'][:_D]
    w1a = p['Wh1'][_D:]
    nb = jnp.zeros((8, _D), jnp.float32)
    nb = nb.at[0].set(p['bh1'])
    nb = nb.at[1].set(p['bh2'])
    full = lambda r, cdim: pl.BlockSpec((r, cdim), lambda i, j: (0, 0))
    out = pl.pallas_call(
        functools.partial(_egnn_body, n, npad, njt),
        grid=(nit, njt),
        in_specs=[
            pl.BlockSpec((_BI, _D), lambda i, j: (i, 0)),
            pl.BlockSpec((_BJ, _D), lambda i, j: (j, 0)),
            pl.BlockSpec((_BI, _D), lambda i, j: (i, 0)),
            pl.BlockSpec((_BJ, _D), lambda i, j: (j, 0)),
            pl.BlockSpec((_BI, _BJ), lambda i, j: (i, j)),
            full(_D, _M),
            full(_D, _M),
            full(_M, _M),
            full(_M, 8),
            full(_D, _D),
            full(_M, _D),
            full(_D, _D),
            full(8, _D),
        ],
        out_specs=pl.BlockSpec((_BI, _D), lambda i, j: (i, 0)),
        out_shape=jax.ShapeDtypeStruct((npad, _D), jnp.float32),
        scratch_shapes=[pltpu.VMEM((_BI, _M), jnp.float32)],
        compiler_params=pltpu.CompilerParams(
            dimension_semantics=("parallel", "arbitrary")),
    )(fp, fp, cp, cp, epd, wi, wj, w2t, sm, w1f, w1a, p['Wh2'], nb)
    return out[:n]


def _pool_body(h_ref, w_ref, b_ref, wsp_ref, out_ref):
    dh = _D // 2
    scale = 1.0 / (dh ** 0.5)
    qkv = _dot(h_ref[...], w_ref[...]) + b_ref[0:1, :]
    q0 = qkv[:, 0:dh]
    q1 = qkv[:, dh:2 * dh]
    k0 = qkv[:, 2 * dh:3 * dh]
    k1 = qkv[:, 3 * dh:4 * dh]
    v0 = qkv[:, 4 * dh:5 * dh]
    v1 = qkv[:, 5 * dh:6 * dh]
    total = None
    for qh in (q0, q1):
        s0 = jnp.sum(qh * k0, axis=1, keepdims=True) * scale
        s1 = jnp.sum(qh * k1, axis=1, keepdims=True) * scale
        mx = jnp.maximum(s0, s1)
        e0 = jnp.exp(s0 - mx)
        e1 = jnp.exp(s1 - mx)
        den = e0 + e1
        oh = (e0 / den) * v0 + (e1 / den) * v1
        hh = 0 if qh is q0 else 1
        sc = _dot(oh, wsp_ref[:, hh:hh + 1])
        total = sc if total is None else total + sc
    out_ref[...] = jnp.broadcast_to(total, out_ref.shape)


def _pool_scores(p, h, n):
    npad = -(-n // _BI) * _BI
    hp = jnp.pad(h, ((0, npad - n), (0, 0)))
    bq = jnp.zeros((8, 3 * _D), jnp.float32).at[0].set(p['bqkv'])
    dh = _D // 2
    wsp = jnp.zeros((dh, _D), jnp.float32)
    wsp = wsp.at[:, 0].set(p['Wsp'][0::2, 0])
    wsp = wsp.at[:, 1].set(p['Wsp'][1::2, 0])
    out = pl.pallas_call(
        _pool_body,
        grid=(npad // _BI,),
        in_specs=[
            pl.BlockSpec((_BI, _D), lambda i: (i, 0)),
            pl.BlockSpec((_D, 3 * _D), lambda i: (0, 0)),
            pl.BlockSpec((8, 3 * _D), lambda i: (0, 0)),
            pl.BlockSpec((dh, _D), lambda i: (0, 0)),
        ],
        out_specs=pl.BlockSpec((_BI, _D), lambda i: (i, 0)),
        out_shape=jax.ShapeDtypeStruct((npad, _D), jnp.float32),
        compiler_params=pltpu.CompilerParams(
            dimension_semantics=("parallel",)),
    )(hp, p['Wqkv'], bq, wsp)
    return out[:n, 0] + p['bsp'][0]


def _adj_body(r_ref, c_ref, o_ref):
    acc = _dot(r_ref[...], c_ref[...])
    o_ref[...] = (acc != 0.0).astype(jnp.float32)


def _adj_apply(rows, cols):
    kk, ninner = rows.shape
    kpad = -(-kk // _BI) * _BI
    ipad = -(-ninner // _BI) * _BI
    rp = jnp.pad(rows, ((0, kpad - kk), (0, ipad - ninner)))
    cp = jnp.pad(cols, ((0, ipad - ninner), (0, kpad - kk)))
    out = pl.pallas_call(
        _adj_body,
        grid=(kpad // _BI, kpad // _BJ),
        in_specs=[
            pl.BlockSpec((_BI, ipad), lambda i, j: (i, 0)),
            pl.BlockSpec((ipad, _BJ), lambda i, j: (0, j)),
        ],
        out_specs=pl.BlockSpec((_BI, _BJ), lambda i, j: (i, j)),
        out_shape=jax.ShapeDtypeStruct((kpad, kpad), jnp.float32),
        compiler_params=pltpu.CompilerParams(
            dimension_semantics=("parallel", "parallel")),
    )(rp, cp)
    return out[:kk, :kk]


def kernel(feat, coor, edge, ep, params):
    f0 = feat[0]
    c0 = coor[0]
    e0 = edge[0, :, :, 0]
    n0 = f0.shape[0]

    h0 = _egnn_apply(params['down'][0], f0, c0, e0, n0)
    s0 = _pool_scores(params['pools'][0], h0, n0)
    k1 = max(2, int(0.8 * n0))
    val0, idx0 = jax.lax.top_k(s0, k1)
    a0 = (e0 != 0).astype(jnp.float32)
    g1 = _adj_apply(a0[idx0, :], a0[:, idx0])
    h1in = h0[idx0] * val0[:, None]
    c1 = c0[idx0]

    h1 = _egnn_apply(params['down'][1], h1in, c1, g1, k1)
    s1 = _pool_scores(params['pools'][1], h1, k1)
    k2 = max(2, int(0.6 * k1))
    val1, idx1 = jax.lax.top_k(s1, k2)
    a1 = (g1 != 0).astype(jnp.float32)
    g2 = _adj_apply(a1[idx1, :], a1[:, idx1])
    h2in = h1[idx1] * val1[:, None]
    c2 = c1[idx1]

    hb = _egnn_apply(params['bottom'], h2in, c2, g2, k2)

    hu1 = jnp.zeros((k1, _D), jnp.float32).at[idx1].set(hb)
    h3 = _egnn_apply(params['up'][0], hu1, c1, g1, k1) + h1
    hu0 = jnp.zeros((n0, _D), jnp.float32).at[idx0].set(h3)
    h4 = _egnn_apply(params['up'][1], hu0, c0, e0, n0) + h0
    return h4


# speedup vs baseline: 1.3514x; 1.3514x over previous
import functools

import jax
import jax.numpy as jnp
from jax.experimental import pallas as pl
from jax.experimental.pallas import tpu as pltpu

_D = 128
_M = 64
_BI = 128
_BJ = 128


def _silu(x):
    return x * (0.5 * jnp.tanh(0.5 * x) + 0.5)


def _dot(a, b, dims=(((1,), (0,)), ((), ()))):
    return jax.lax.dot_general(a, b, dims, preferred_element_type=jnp.float32)


def _egnn_body(n, npad, njt, fi_ref, fj_ref, ci_ref, cj_ref, e_ref, wi_ref,
               wj_ref, w2t_ref, sm_ref, w1f_ref, w1a_ref, wh2_ref, nb_ref,
               out_ref):
    fi = fi_ref[...]
    ci = ci_ref[...]
    fj = fj_ref[...]
    cj = cj_ref[...]
    wdc = sm_ref[:, 0:1]
    wec = sm_ref[:, 1:2]
    be1c = sm_ref[:, 2:3]
    be2c = sm_ref[:, 3:4]

    ti = _dot(fi, wi_ref[...])
    tjT = jax.lax.dot_general(wj_ref[...], fj, (((0,), (1,)), ((), ())),
                              preferred_element_type=jnp.float32)
    qi = jnp.sum(ci * ci, axis=1, keepdims=True)
    qjT = jax.lax.dot_general(jnp.ones((1, _D), jnp.float32), cj * cj,
                              (((1,), (1,)), ((), ())),
                              preferred_element_type=jnp.float32)
    cross = jax.lax.dot_general(ci, cj, (((1,), (1,)), ((), ())),
                                preferred_element_type=jnp.float32)
    vT = tjT + wdc * qjT + be1c
    scal = qi - 2.0 * cross
    ti3 = ti[:, :, None]
    w2tb = jnp.broadcast_to(w2t_ref[...][None], (_BI, _M, _M))

    agg = jnp.zeros((_BI, _M), jnp.float32)
    for t in range(njt):
        sl = slice(t * _BJ, (t + 1) * _BJ)
        ee = e_ref[:, sl]
        m = (ti3 + vT[None, :, sl]
             + scal[:, None, sl] * wdc[None, :, :]
             + ee[:, None, :] * wec[None, :, :])
        m = _silu(m)
        m = jax.lax.dot_general(w2tb, m, (((2,), (1,)), ((0,), (0,))),
                                preferred_element_type=jnp.float32)
        m = _silu(m + be2c[None, :, :])
        if t == njt - 1 and npad != n:
            jj = (t * _BJ
                  + jax.lax.broadcasted_iota(jnp.int32, (1, 1, _BJ), 2))
            m = jnp.where(jj < n, m, 0.0)
        agg = agg + jnp.sum(m, axis=2)

    f = fi
    h1 = _silu(_dot(f, w1f_ref[...]) + _dot(agg, w1a_ref[...])
               + nb_ref[0:1, :])
    out = _dot(h1, wh2_ref[...]) + nb_ref[1:2, :]
    out_ref[...] = jnp.maximum(f + out, 0.0)


def _egnn_apply(p, f, c, e, n):
    npad = -(-n // _BI) * _BI
    nit = npad // _BI
    njt = npad // _BJ
    fp = jnp.pad(f, ((0, npad - n), (0, 0)))
    cp = jnp.pad(c, ((0, npad - n), (0, _D - c.shape[1])))
    epd = jnp.pad(e, ((0, npad - n), (0, npad - n)))
    wi = p['We1'][:_D]
    wj = p['We1'][_D:2 * _D]
    sm = jnp.zeros((_M, 8), jnp.float32)
    sm = sm.at[:, 0].set(p['We1'][2 * _D])
    sm = sm.at[:, 1].set(p['We1'][2 * _D + 1])
    sm = sm.at[:, 2].set(p['be1'])
    sm = sm.at[:, 3].set(p['be2'])
    w2t = p['We2'].T
    w1f = p['Wh1'][:_D]
    w1a = p['Wh1'][_D:]
    nb = jnp.zeros((8, _D), jnp.float32)
    nb = nb.at[0].set(p['bh1'])
    nb = nb.at[1].set(p['bh2'])
    full = lambda r, cdim: pl.BlockSpec((r, cdim), lambda i: (0, 0))
    out = pl.pallas_call(
        functools.partial(_egnn_body, n, npad, njt),
        grid=(nit,),
        in_specs=[
            pl.BlockSpec((_BI, _D), lambda i: (i, 0)),
            pl.BlockSpec((npad, _D), lambda i: (0, 0)),
            pl.BlockSpec((_BI, _D), lambda i: (i, 0)),
            pl.BlockSpec((npad, _D), lambda i: (0, 0)),
            pl.BlockSpec((_BI, npad), lambda i: (i, 0)),
            full(_D, _M),
            full(_D, _M),
            full(_M, _M),
            full(_M, 8),
            full(_D, _D),
            full(_M, _D),
            full(_D, _D),
            full(8, _D),
        ],
        out_specs=pl.BlockSpec((_BI, _D), lambda i: (i, 0)),
        out_shape=jax.ShapeDtypeStruct((npad, _D), jnp.float32),
        compiler_params=pltpu.CompilerParams(
            dimension_semantics=("arbitrary",)),
    )(fp, fp, cp, cp, epd, wi, wj, w2t, sm, w1f, w1a, p['Wh2'], nb)
    return out[:n]


def _pool_body(h_ref, w_ref, b_ref, wsp_ref, out_ref):
    dh = _D // 2
    scale = 1.0 / (dh ** 0.5)
    qkv = _dot(h_ref[...], w_ref[...]) + b_ref[0:1, :]
    q0 = qkv[:, 0:dh]
    q1 = qkv[:, dh:2 * dh]
    k0 = qkv[:, 2 * dh:3 * dh]
    k1 = qkv[:, 3 * dh:4 * dh]
    v0 = qkv[:, 4 * dh:5 * dh]
    v1 = qkv[:, 5 * dh:6 * dh]
    total = None
    for qh in (q0, q1):
        s0 = jnp.sum(qh * k0, axis=1, keepdims=True) * scale
        s1 = jnp.sum(qh * k1, axis=1, keepdims=True) * scale
        mx = jnp.maximum(s0, s1)
        e0 = jnp.exp(s0 - mx)
        e1 = jnp.exp(s1 - mx)
        den = e0 + e1
        oh = (e0 / den) * v0 + (e1 / den) * v1
        hh = 0 if qh is q0 else 1
        sc = _dot(oh, wsp_ref[:, hh:hh + 1])
        total = sc if total is None else total + sc
    out_ref[...] = jnp.broadcast_to(total, out_ref.shape)


def _pool_scores(p, h, n):
    npad = -(-n // _BI) * _BI
    hp = jnp.pad(h, ((0, npad - n), (0, 0)))
    bq = jnp.zeros((8, 3 * _D), jnp.float32).at[0].set(p['bqkv'])
    dh = _D // 2
    wsp = jnp.zeros((dh, _D), jnp.float32)
    wsp = wsp.at[:, 0].set(p['Wsp'][0::2, 0])
    wsp = wsp.at[:, 1].set(p['Wsp'][1::2, 0])
    out = pl.pallas_call(
        _pool_body,
        grid=(npad // _BI,),
        in_specs=[
            pl.BlockSpec((_BI, _D), lambda i: (i, 0)),
            pl.BlockSpec((_D, 3 * _D), lambda i: (0, 0)),
            pl.BlockSpec((8, 3 * _D), lambda i: (0, 0)),
            pl.BlockSpec((dh, _D), lambda i: (0, 0)),
        ],
        out_specs=pl.BlockSpec((_BI, _D), lambda i: (i, 0)),
        out_shape=jax.ShapeDtypeStruct((npad, _D), jnp.float32),
        compiler_params=pltpu.CompilerParams(
            dimension_semantics=("parallel",)),
    )(hp, p['Wqkv'], bq, wsp)
    return out[:n, 0] + p['bsp'][0]


def _adj_body(r_ref, c_ref, o_ref):
    acc = _dot(r_ref[...], c_ref[...])
    o_ref[...] = (acc != 0.0).astype(jnp.float32)


def _adj_apply(rows, cols):
    kk, ninner = rows.shape
    kpad = -(-kk // _BI) * _BI
    ipad = -(-ninner // _BI) * _BI
    rp = jnp.pad(rows, ((0, kpad - kk), (0, ipad - ninner)))
    cp = jnp.pad(cols, ((0, ipad - ninner), (0, kpad - kk)))
    out = pl.pallas_call(
        _adj_body,
        grid=(kpad // _BI, kpad // _BJ),
        in_specs=[
            pl.BlockSpec((_BI, ipad), lambda i, j: (i, 0)),
            pl.BlockSpec((ipad, _BJ), lambda i, j: (0, j)),
        ],
        out_specs=pl.BlockSpec((_BI, _BJ), lambda i, j: (i, j)),
        out_shape=jax.ShapeDtypeStruct((kpad, kpad), jnp.float32),
        compiler_params=pltpu.CompilerParams(
            dimension_semantics=("parallel", "parallel")),
    )(rp, cp)
    return out[:kk, :kk]


def kernel(feat, coor, edge, ep, params):
    f0 = feat[0]
    c0 = coor[0]
    e0 = edge[0, :, :, 0]
    n0 = f0.shape[0]

    h0 = _egnn_apply(params['down'][0], f0, c0, e0, n0)
    s0 = _pool_scores(params['pools'][0], h0, n0)
    k1 = max(2, int(0.8 * n0))
    val0, idx0 = jax.lax.top_k(s0, k1)
    a0 = (e0 != 0).astype(jnp.float32)
    g1 = _adj_apply(a0[idx0, :], a0[:, idx0])
    h1in = h0[idx0] * val0[:, None]
    c1 = c0[idx0]

    h1 = _egnn_apply(params['down'][1], h1in, c1, g1, k1)
    s1 = _pool_scores(params['pools'][1], h1, k1)
    k2 = max(2, int(0.6 * k1))
    val1, idx1 = jax.lax.top_k(s1, k2)
    a1 = (g1 != 0).astype(jnp.float32)
    g2 = _adj_apply(a1[idx1, :], a1[:, idx1])
    h2in = h1[idx1] * val1[:, None]
    c2 = c1[idx1]

    hb = _egnn_apply(params['bottom'], h2in, c2, g2, k2)

    hu1 = jnp.zeros((k1, _D), jnp.float32).at[idx1].set(hb)
    h3 = _egnn_apply(params['up'][0], hu1, c1, g1, k1) + h1
    hu0 = jnp.zeros((n0, _D), jnp.float32).at[idx0].set(h3)
    h4 = _egnn_apply(params['up'][1], hu0, c0, e0, n0) + h0
    return h4


# speedup vs baseline: 1.6526x; 1.2229x over previous
import functools

import jax
import jax.numpy as jnp
from jax.experimental import pallas as pl
from jax.experimental.pallas import tpu as pltpu

_D = 128
_M = 64
_BI = 128
_BJ = 128


def _silu(x):
    return x * (0.5 * jnp.tanh(0.5 * x) + 0.5)


def _hsilu(y):
    t = jnp.tanh(y)
    return y * t + y


def _dot(a, b, dims=(((1,), (0,)), ((), ()))):
    return jax.lax.dot_general(a, b, dims, preferred_element_type=jnp.float32)


def _egnn_body(n, npad, njt, fi_ref, fj_ref, ci_ref, cj_ref, e_ref, wi_ref,
               wj_ref, w2t_ref, sm_ref, w1f_ref, w1a_ref, wh2_ref, nb_ref,
               out_ref):
    fi = fi_ref[...]
    ci = ci_ref[...]
    fj = fj_ref[...]
    cj = cj_ref[...]
    wdc = sm_ref[:, 0:1]
    wec = sm_ref[:, 1:2]
    be1c = sm_ref[:, 2:3]
    be2c = sm_ref[:, 3:4]

    ti = _dot(fi, wi_ref[...])
    tjT = jax.lax.dot_general(wj_ref[...], fj, (((0,), (1,)), ((), ())),
                              preferred_element_type=jnp.float32)
    qi = jnp.sum(ci * ci, axis=1, keepdims=True)
    qjT = jax.lax.dot_general(jnp.ones((1, _D), jnp.float32), cj * cj,
                              (((1,), (1,)), ((), ())),
                              preferred_element_type=jnp.float32)
    cross = jax.lax.dot_general(ci, cj, (((1,), (1,)), ((), ())),
                                preferred_element_type=jnp.float32)
    vT = tjT + wdc * qjT + be1c
    scal = qi - 2.0 * cross
    ti3 = ti[:, :, None]
    w2tb = jnp.broadcast_to(w2t_ref[...][None], (_BI, _M, _M))

    agg = jnp.zeros((_BI, _M), jnp.float32)
    for t in range(njt):
        sl = slice(t * _BJ, (t + 1) * _BJ)
        ee = e_ref[:, sl]
        m = (ti3 + vT[None, :, sl]
             + scal[:, None, sl] * wdc[None, :, :]
             + ee[:, None, :] * wec[None, :, :])
        m = _hsilu(m)
        m = jax.lax.dot_general(w2tb, m, (((2,), (1,)), ((0,), (0,))),
                                preferred_element_type=jnp.float32)
        m = _hsilu(m + be2c[None, :, :])
        if t == njt - 1 and npad != n:
            jj = (t * _BJ
                  + jax.lax.broadcasted_iota(jnp.int32, (1, 1, _BJ), 2))
            m = jnp.where(jj < n, m, 0.0)
        agg = agg + jnp.sum(m, axis=2)

    f = fi
    h1 = _silu(_dot(f, w1f_ref[...]) + _dot(agg, w1a_ref[...])
               + nb_ref[0:1, :])
    out = _dot(h1, wh2_ref[...]) + nb_ref[1:2, :]
    out_ref[...] = jnp.maximum(f + out, 0.0)


def _egnn_apply(p, f, c, e, n):
    npad = -(-n // _BI) * _BI
    nit = npad // _BI
    njt = npad // _BJ
    fp = jnp.pad(f, ((0, npad - n), (0, 0)))
    cp = jnp.pad(c, ((0, npad - n), (0, _D - c.shape[1])))
    epd = jnp.pad(e, ((0, npad - n), (0, npad - n)))
    wi = 0.5 * p['We1'][:_D]
    wj = 0.5 * p['We1'][_D:2 * _D]
    sm = jnp.zeros((_M, 8), jnp.float32)
    sm = sm.at[:, 0].set(0.5 * p['We1'][2 * _D])
    sm = sm.at[:, 1].set(0.5 * p['We1'][2 * _D + 1])
    sm = sm.at[:, 2].set(0.5 * p['be1'])
    sm = sm.at[:, 3].set(0.5 * p['be2'])
    w2t = 0.5 * p['We2'].T
    w1f = p['Wh1'][:_D]
    w1a = p['Wh1'][_D:]
    nb = jnp.zeros((8, _D), jnp.float32)
    nb = nb.at[0].set(p['bh1'])
    nb = nb.at[1].set(p['bh2'])
    full = lambda r, cdim: pl.BlockSpec((r, cdim), lambda i: (0, 0))
    out = pl.pallas_call(
        functools.partial(_egnn_body, n, npad, njt),
        grid=(nit,),
        in_specs=[
            pl.BlockSpec((_BI, _D), lambda i: (i, 0)),
            pl.BlockSpec((npad, _D), lambda i: (0, 0)),
            pl.BlockSpec((_BI, _D), lambda i: (i, 0)),
            pl.BlockSpec((npad, _D), lambda i: (0, 0)),
            pl.BlockSpec((_BI, npad), lambda i: (i, 0)),
            full(_D, _M),
            full(_D, _M),
            full(_M, _M),
            full(_M, 8),
            full(_D, _D),
            full(_M, _D),
            full(_D, _D),
            full(8, _D),
        ],
        out_specs=pl.BlockSpec((_BI, _D), lambda i: (i, 0)),
        out_shape=jax.ShapeDtypeStruct((npad, _D), jnp.float32),
        compiler_params=pltpu.CompilerParams(
            dimension_semantics=("arbitrary",)),
    )(fp, fp, cp, cp, epd, wi, wj, w2t, sm, w1f, w1a, p['Wh2'], nb)
    return out[:n]


def _pool_body(h_ref, w_ref, b_ref, wsp_ref, out_ref):
    dh = _D // 2
    scale = 1.0 / (dh ** 0.5)
    qkv = _dot(h_ref[...], w_ref[...]) + b_ref[0:1, :]
    q0 = qkv[:, 0:dh]
    q1 = qkv[:, dh:2 * dh]
    k0 = qkv[:, 2 * dh:3 * dh]
    k1 = qkv[:, 3 * dh:4 * dh]
    v0 = qkv[:, 4 * dh:5 * dh]
    v1 = qkv[:, 5 * dh:6 * dh]
    total = None
    for qh in (q0, q1):
        s0 = jnp.sum(qh * k0, axis=1, keepdims=True) * scale
        s1 = jnp.sum(qh * k1, axis=1, keepdims=True) * scale
        mx = jnp.maximum(s0, s1)
        e0 = jnp.exp(s0 - mx)
        e1 = jnp.exp(s1 - mx)
        den = e0 + e1
        oh = (e0 / den) * v0 + (e1 / den) * v1
        hh = 0 if qh is q0 else 1
        sc = _dot(oh, wsp_ref[:, hh:hh + 1])
        total = sc if total is None else total + sc
    out_ref[...] = jnp.broadcast_to(total, out_ref.shape)


def _pool_scores(p, h, n):
    npad = -(-n // _BI) * _BI
    hp = jnp.pad(h, ((0, npad - n), (0, 0)))
    bq = jnp.zeros((8, 3 * _D), jnp.float32).at[0].set(p['bqkv'])
    dh = _D // 2
    wsp = jnp.zeros((dh, _D), jnp.float32)
    wsp = wsp.at[:, 0].set(p['Wsp'][0::2, 0])
    wsp = wsp.at[:, 1].set(p['Wsp'][1::2, 0])
    out = pl.pallas_call(
        _pool_body,
        grid=(npad // _BI,),
        in_specs=[
            pl.BlockSpec((_BI, _D), lambda i: (i, 0)),
            pl.BlockSpec((_D, 3 * _D), lambda i: (0, 0)),
            pl.BlockSpec((8, 3 * _D), lambda i: (0, 0)),
            pl.BlockSpec((dh, _D), lambda i: (0, 0)),
        ],
        out_specs=pl.BlockSpec((_BI, _D), lambda i: (i, 0)),
        out_shape=jax.ShapeDtypeStruct((npad, _D), jnp.float32),
        compiler_params=pltpu.CompilerParams(
            dimension_semantics=("parallel",)),
    )(hp, p['Wqkv'], bq, wsp)
    return out[:n, 0] + p['bsp'][0]


def _adj_body(r_ref, c_ref, o_ref):
    acc = _dot(r_ref[...], c_ref[...])
    o_ref[...] = (acc != 0.0).astype(jnp.float32)


def _adj_apply(rows, cols):
    kk, ninner = rows.shape
    kpad = -(-kk // _BI) * _BI
    ipad = -(-ninner // _BI) * _BI
    rp = jnp.pad(rows, ((0, kpad - kk), (0, ipad - ninner)))
    cp = jnp.pad(cols, ((0, ipad - ninner), (0, kpad - kk)))
    out = pl.pallas_call(
        _adj_body,
        grid=(kpad // _BI, kpad // _BJ),
        in_specs=[
            pl.BlockSpec((_BI, ipad), lambda i, j: (i, 0)),
            pl.BlockSpec((ipad, _BJ), lambda i, j: (0, j)),
        ],
        out_specs=pl.BlockSpec((_BI, _BJ), lambda i, j: (i, j)),
        out_shape=jax.ShapeDtypeStruct((kpad, kpad), jnp.float32),
        compiler_params=pltpu.CompilerParams(
            dimension_semantics=("parallel", "parallel")),
    )(rp, cp)
    return out[:kk, :kk]


def kernel(feat, coor, edge, ep, params):
    f0 = feat[0]
    c0 = coor[0]
    e0 = edge[0, :, :, 0]
    n0 = f0.shape[0]

    h0 = _egnn_apply(params['down'][0], f0, c0, e0, n0)
    s0 = _pool_scores(params['pools'][0], h0, n0)
    k1 = max(2, int(0.8 * n0))
    val0, idx0 = jax.lax.top_k(s0, k1)
    a0 = (e0 != 0).astype(jnp.float32)
    g1 = _adj_apply(a0[idx0, :], a0[:, idx0])
    h1in = h0[idx0] * val0[:, None]
    c1 = c0[idx0]

    h1 = _egnn_apply(params['down'][1], h1in, c1, g1, k1)
    s1 = _pool_scores(params['pools'][1], h1, k1)
    k2 = max(2, int(0.6 * k1))
    val1, idx1 = jax.lax.top_k(s1, k2)
    a1 = (g1 != 0).astype(jnp.float32)
    g2 = _adj_apply(a1[idx1, :], a1[:, idx1])
    h2in = h1[idx1] * val1[:, None]
    c2 = c1[idx1]

    hb = _egnn_apply(params['bottom'], h2in, c2, g2, k2)

    hu1 = jnp.zeros((k1, _D), jnp.float32).at[idx1].set(hb)
    h3 = _egnn_apply(params['up'][0], hu1, c1, g1, k1) + h1
    hu0 = jnp.zeros((n0, _D), jnp.float32).at[idx0].set(h3)
    h4 = _egnn_apply(params['up'][1], hu0, c0, e0, n0) + h0
    return h4


# speedup vs baseline: 1.8004x; 1.0895x over previous
import functools

import jax
import jax.numpy as jnp
from jax.experimental import pallas as pl
from jax.experimental.pallas import tpu as pltpu

_D = 128
_M = 64
_BI = 128
_BJ = 128


def _silu(x):
    return x * (0.5 * jnp.tanh(0.5 * x) + 0.5)


def _hsilu(y):
    t = jnp.tanh(y)
    return y * t + y


def _dot(a, b, dims=(((1,), (0,)), ((), ()))):
    return jax.lax.dot_general(a, b, dims, preferred_element_type=jnp.float32)


def _egnn_body(n, npad, njt, fi_ref, fj_ref, ci_ref, cj_ref, e_ref, wi_ref,
               wj_ref, w2t_ref, sm_ref, w1f_ref, w1a_ref, wh2_ref, nb_ref,
               out_ref):
    fi = fi_ref[...]
    ci = ci_ref[...]
    fj = fj_ref[...]
    cj = cj_ref[...]
    wdc = sm_ref[:, 0:1]
    wec = sm_ref[:, 1:2]
    be1c = sm_ref[:, 2:3]
    be2c = sm_ref[:, 3:4]

    ti = _dot(fi, wi_ref[...])
    tjT = jax.lax.dot_general(wj_ref[...], fj, (((0,), (1,)), ((), ())),
                              preferred_element_type=jnp.float32)
    qi = jnp.sum(ci * ci, axis=1, keepdims=True)
    qjT = jax.lax.dot_general(jnp.ones((1, _D), jnp.float32), cj * cj,
                              (((1,), (1,)), ((), ())),
                              preferred_element_type=jnp.float32)
    cross = jax.lax.dot_general(ci, cj, (((1,), (1,)), ((), ())),
                                preferred_element_type=jnp.float32)
    vT = tjT + wdc * qjT + be1c
    w2tb = jnp.broadcast_to(w2t_ref[...][None], (_BI, _M, _M))
    w3b = jnp.concatenate([
        jnp.broadcast_to((-2.0 * wdc)[None], (_BI, _M, 1)),
        jnp.broadcast_to(wec[None], (_BI, _M, 1)),
        ti[:, :, None],
        jnp.broadcast_to(wdc[None], (_BI, _M, 1)),
    ], axis=2)
    ones_row = jnp.ones((_BI, 1, _BJ), jnp.float32)
    qi3 = jnp.broadcast_to(qi[:, :, None], (_BI, 1, _BJ))

    agg = jnp.zeros((_BI, _M), jnp.float32)
    for t in range(njt):
        sl = slice(t * _BJ, (t + 1) * _BJ)
        s3 = jnp.concatenate([
            cross[:, None, sl],
            e_ref[:, sl][:, None, :],
            ones_row,
            qi3,
        ], axis=1)
        m = jax.lax.dot_general(w3b, s3, (((2,), (1,)), ((0,), (0,))),
                                preferred_element_type=jnp.float32)
        m = _hsilu(m + vT[None, :, sl])
        m = jax.lax.dot_general(w2tb, m, (((2,), (1,)), ((0,), (0,))),
                                preferred_element_type=jnp.float32)
        m = _hsilu(m + be2c[None, :, :])
        if t == njt - 1 and npad != n:
            jj = (t * _BJ
                  + jax.lax.broadcasted_iota(jnp.int32, (1, 1, _BJ), 2))
            m = jnp.where(jj < n, m, 0.0)
        agg = agg + jnp.sum(m, axis=2)

    f = fi
    h1 = _silu(_dot(f, w1f_ref[...]) + _dot(agg, w1a_ref[...])
               + nb_ref[0:1, :])
    out = _dot(h1, wh2_ref[...]) + nb_ref[1:2, :]
    out_ref[...] = jnp.maximum(f + out, 0.0)


def _egnn_apply(p, f, c, e, n):
    npad = -(-n // _BI) * _BI
    nit = npad // _BI
    njt = npad // _BJ
    fp = jnp.pad(f, ((0, npad - n), (0, 0)))
    cp = jnp.pad(c, ((0, npad - n), (0, _D - c.shape[1])))
    epd = jnp.pad(e, ((0, npad - n), (0, npad - n)))
    wi = 0.5 * p['We1'][:_D]
    wj = 0.5 * p['We1'][_D:2 * _D]
    sm = jnp.zeros((_M, 8), jnp.float32)
    sm = sm.at[:, 0].set(0.5 * p['We1'][2 * _D])
    sm = sm.at[:, 1].set(0.5 * p['We1'][2 * _D + 1])
    sm = sm.at[:, 2].set(0.5 * p['be1'])
    sm = sm.at[:, 3].set(0.5 * p['be2'])
    w2t = 0.5 * p['We2'].T
    w1f = p['Wh1'][:_D]
    w1a = p['Wh1'][_D:]
    nb = jnp.zeros((8, _D), jnp.float32)
    nb = nb.at[0].set(p['bh1'])
    nb = nb.at[1].set(p['bh2'])
    full = lambda r, cdim: pl.BlockSpec((r, cdim), lambda i: (0, 0))
    out = pl.pallas_call(
        functools.partial(_egnn_body, n, npad, njt),
        grid=(nit,),
        in_specs=[
            pl.BlockSpec((_BI, _D), lambda i: (i, 0)),
            pl.BlockSpec((npad, _D), lambda i: (0, 0)),
            pl.BlockSpec((_BI, _D), lambda i: (i, 0)),
            pl.BlockSpec((npad, _D), lambda i: (0, 0)),
            pl.BlockSpec((_BI, npad), lambda i: (i, 0)),
            full(_D, _M),
            full(_D, _M),
            full(_M, _M),
            full(_M, 8),
            full(_D, _D),
            full(_M, _D),
            full(_D, _D),
            full(8, _D),
        ],
        out_specs=pl.BlockSpec((_BI, _D), lambda i: (i, 0)),
        out_shape=jax.ShapeDtypeStruct((npad, _D), jnp.float32),
        compiler_params=pltpu.CompilerParams(
            dimension_semantics=("arbitrary",)),
    )(fp, fp, cp, cp, epd, wi, wj, w2t, sm, w1f, w1a, p['Wh2'], nb)
    return out[:n]


def _pool_body(h_ref, w_ref, b_ref, wsp_ref, out_ref):
    dh = _D // 2
    scale = 1.0 / (dh ** 0.5)
    qkv = _dot(h_ref[...], w_ref[...]) + b_ref[0:1, :]
    q0 = qkv[:, 0:dh]
    q1 = qkv[:, dh:2 * dh]
    k0 = qkv[:, 2 * dh:3 * dh]
    k1 = qkv[:, 3 * dh:4 * dh]
    v0 = qkv[:, 4 * dh:5 * dh]
    v1 = qkv[:, 5 * dh:6 * dh]
    total = None
    for qh in (q0, q1):
        s0 = jnp.sum(qh * k0, axis=1, keepdims=True) * scale
        s1 = jnp.sum(qh * k1, axis=1, keepdims=True) * scale
        mx = jnp.maximum(s0, s1)
        e0 = jnp.exp(s0 - mx)
        e1 = jnp.exp(s1 - mx)
        den = e0 + e1
        oh = (e0 / den) * v0 + (e1 / den) * v1
        hh = 0 if qh is q0 else 1
        sc = _dot(oh, wsp_ref[:, hh:hh + 1])
        total = sc if total is None else total + sc
    out_ref[...] = jnp.broadcast_to(total, out_ref.shape)


def _pool_scores(p, h, n):
    npad = -(-n // _BI) * _BI
    hp = jnp.pad(h, ((0, npad - n), (0, 0)))
    bq = jnp.zeros((8, 3 * _D), jnp.float32).at[0].set(p['bqkv'])
    dh = _D // 2
    wsp = jnp.zeros((dh, _D), jnp.float32)
    wsp = wsp.at[:, 0].set(p['Wsp'][0::2, 0])
    wsp = wsp.at[:, 1].set(p['Wsp'][1::2, 0])
    out = pl.pallas_call(
        _pool_body,
        grid=(npad // _BI,),
        in_specs=[
            pl.BlockSpec((_BI, _D), lambda i: (i, 0)),
            pl.BlockSpec((_D, 3 * _D), lambda i: (0, 0)),
            pl.BlockSpec((8, 3 * _D), lambda i: (0, 0)),
            pl.BlockSpec((dh, _D), lambda i: (0, 0)),
        ],
        out_specs=pl.BlockSpec((_BI, _D), lambda i: (i, 0)),
        out_shape=jax.ShapeDtypeStruct((npad, _D), jnp.float32),
        compiler_params=pltpu.CompilerParams(
            dimension_semantics=("parallel",)),
    )(hp, p['Wqkv'], bq, wsp)
    return out[:n, 0] + p['bsp'][0]


def _adj_body(r_ref, c_ref, o_ref):
    acc = _dot(r_ref[...], c_ref[...])
    o_ref[...] = (acc != 0.0).astype(jnp.float32)


def _adj_apply(rows, cols):
    kk, ninner = rows.shape
    kpad = -(-kk // _BI) * _BI
    ipad = -(-ninner // _BI) * _BI
    rp = jnp.pad(rows, ((0, kpad - kk), (0, ipad - ninner)))
    cp = jnp.pad(cols, ((0, ipad - ninner), (0, kpad - kk)))
    out = pl.pallas_call(
        _adj_body,
        grid=(kpad // _BI, kpad // _BJ),
        in_specs=[
            pl.BlockSpec((_BI, ipad), lambda i, j: (i, 0)),
            pl.BlockSpec((ipad, _BJ), lambda i, j: (0, j)),
        ],
        out_specs=pl.BlockSpec((_BI, _BJ), lambda i, j: (i, j)),
        out_shape=jax.ShapeDtypeStruct((kpad, kpad), jnp.float32),
        compiler_params=pltpu.CompilerParams(
            dimension_semantics=("parallel", "parallel")),
    )(rp, cp)
    return out[:kk, :kk]


def kernel(feat, coor, edge, ep, params):
    f0 = feat[0]
    c0 = coor[0]
    e0 = edge[0, :, :, 0]
    n0 = f0.shape[0]

    h0 = _egnn_apply(params['down'][0], f0, c0, e0, n0)
    s0 = _pool_scores(params['pools'][0], h0, n0)
    k1 = max(2, int(0.8 * n0))
    val0, idx0 = jax.lax.top_k(s0, k1)
    a0 = (e0 != 0).astype(jnp.float32)
    g1 = _adj_apply(a0[idx0, :], a0[:, idx0])
    h1in = h0[idx0] * val0[:, None]
    c1 = c0[idx0]

    h1 = _egnn_apply(params['down'][1], h1in, c1, g1, k1)
    s1 = _pool_scores(params['pools'][1], h1, k1)
    k2 = max(2, int(0.6 * k1))
    val1, idx1 = jax.lax.top_k(s1, k2)
    a1 = (g1 != 0).astype(jnp.float32)
    g2 = _adj_apply(a1[idx1, :], a1[:, idx1])
    h2in = h1[idx1] * val1[:, None]
    c2 = c1[idx1]

    hb = _egnn_apply(params['bottom'], h2in, c2, g2, k2)

    hu1 = jnp.zeros((k1, _D), jnp.float32).at[idx1].set(hb)
    h3 = _egnn_apply(params['up'][0], hu1, c1, g1, k1) + h1
    hu0 = jnp.zeros((n0, _D), jnp.float32).at[idx0].set(h3)
    h4 = _egnn_apply(params['up'][1], hu0, c0, e0, n0) + h0
    return h4
